# baseline (speedup 1.0000x reference)
import jax
import jax.numpy as jnp
from jax import lax
from jax.experimental import pallas as pl
from jax.experimental.pallas import tpu as pltpu

_NBLK = 24
_DILC = 8


def _decoder_kernel(x_ref, num_ref, cat_ref, embt_ref, win_ref, bin_ref,
                    q_ref, wc0_ref, wc1_ref, bc_ref, wrt_ref, brt_ref,
                    wst_ref, bst_ref, wo1_ref, bo1_ref, wo2_ref, bo2_ref,
                    out_ref, newq_hbm, tr_ref, cur_ref, acc_ref, sems):
    i = pl.program_id(0)
    f32 = jnp.float32
    B = x_ref.shape[0]
    C = cur_ref.shape[0]
    Tq = q_ref.shape[2] // C

    @pl.when(i == 0)
    def _init():
        xT = jnp.transpose(x_ref[:, :, 0], (1, 0))
        numT = jnp.transpose(num_ref[:, :, 0], (1, 0))
        catT = jnp.transpose(cat_ref[:, :, 0], (1, 0))
        ohT = (lax.broadcasted_iota(jnp.int32, (1000, 1), 0) == catT).astype(f32)
        embT = jnp.dot(embt_ref[...], ohT, preferred_element_type=f32)
        w = win_ref[...]
        cur0 = (w[:, 0:1] * xT
                + jnp.dot(w[:, 1:9], numT, preferred_element_type=f32)
                + jnp.dot(w[:, 9:25], embT, preferred_element_type=f32)
                + bin_ref[...])
        cur_ref[...] = cur0
        acc_ref[...] = jnp.zeros_like(acc_ref)

    qT = jnp.transpose(q_ref[0], (1, 0))
    tr_ref[0:Tq] = jnp.transpose(qT.reshape(C, Tq, B), (1, 0, 2))
    cur = cur_ref[...]
    tr_ref[Tq] = cur

    def _tap(d):
        return lambda: tr_ref[Tq - d]

    tapT = lax.switch(jnp.bitwise_and(i, _DILC - 1),
                      [_tap(1 << k) for k in range(_DILC)])

    zT = (jnp.dot(wc0_ref[0], tapT, preferred_element_type=f32)
          + jnp.dot(wc1_ref[0], cur, preferred_element_type=f32)
          + bc_ref[0])
    fz = jnp.tanh(zT[:C, :])
    gz = zT[C:, :]
    gatedT = fz / (1.0 + jnp.exp(-gz))

    skipT = jnp.dot(wst_ref[0], gatedT, preferred_element_type=f32) + bst_ref[0]
    acc_ref[...] += jnp.dot(wo1_ref[0], jnp.maximum(skipT, 0.0),
                            preferred_element_type=f32)
    cur_ref[...] = (jnp.dot(wrt_ref[0], gatedT, preferred_element_type=f32)
                    + brt_ref[0] + cur)

    cp = pltpu.make_async_copy(tr_ref, newq_hbm.at[i], sems)
    cp.start()
    cp.wait()

    @pl.when(i == _NBLK - 1)
    def _finalize():
        hT = jnp.maximum(acc_ref[...] + bo1_ref[...], 0.0)
        outT = jnp.dot(wo2_ref[...], hT, preferred_element_type=f32) + bo2_ref[...]
        out_ref[...] = outT[None]


def kernel(queues, x, num, cat, emb_table, W_in, b_in, W_conv, b_conv,
           W_res, b_res, W_skip, b_skip, W_o1, b_o1, W_o2, b_o2):
    B, C, Tq = queues.shape[1], queues.shape[2], queues.shape[3]
    S = W_skip.shape[1]

    q2 = queues.reshape(_NBLK, B, C * Tq)
    wc0 = W_conv[:, :, :, 0]
    wc1 = W_conv[:, :, :, 1]
    bc = b_conv[:, :, None]
    brt = b_res[:, :, None]
    bst = b_skip[:, :, None]
    wo1 = W_o1.reshape(128, _NBLK, S).transpose(1, 0, 2)
    binr = b_in[:, None]
    bo1 = b_o1[:, None]
    bo2 = b_o2[:, None]
    embt = emb_table.T
    cat32 = cat.astype(jnp.int32)

    bspec = pl.BlockSpec

    out, newq = pl.pallas_call(
        _decoder_kernel,
        grid=(_NBLK,),
        in_specs=[
            bspec((B, 1, 1), lambda i: (0, 0, 0)),
            bspec((B, 8, 1), lambda i: (0, 0, 0)),
            bspec((B, 1, 1), lambda i: (0, 0, 0)),
            bspec((16, 1000), lambda i: (0, 0)),
            bspec((C, 25), lambda i: (0, 0)),
            bspec((C, 1), lambda i: (0, 0)),
            bspec((1, B, C * Tq), lambda i: (i, 0, 0)),
            bspec((1, 2 * C, C), lambda i: (i, 0, 0)),
            bspec((1, 2 * C, C), lambda i: (i, 0, 0)),
            bspec((1, 2 * C, 1), lambda i: (i, 0, 0)),
            bspec((1, C, C), lambda i: (i, 0, 0)),
            bspec((1, C, 1), lambda i: (i, 0, 0)),
            bspec((1, S, C), lambda i: (i, 0, 0)),
            bspec((1, S, 1), lambda i: (i, 0, 0)),
            bspec((1, 128, S), lambda i: (i, 0, 0)),
            bspec((128, 1), lambda i: (0, 0)),
            bspec((1, 128), lambda i: (0, 0)),
            bspec((1, 1), lambda i: (0, 0)),
        ],
        out_specs=[
            bspec((1, 1, B), lambda i: (0, 0, 0)),
            bspec(memory_space=pl.ANY),
        ],
        out_shape=[
            jax.ShapeDtypeStruct((1, 1, B), jnp.float32),
            jax.ShapeDtypeStruct((_NBLK, Tq + 1, C, B), jnp.float32),
        ],
        scratch_shapes=[
            pltpu.VMEM((Tq + 1, C, B), jnp.float32),
            pltpu.VMEM((C, B), jnp.float32),
            pltpu.VMEM((128, B), jnp.float32),
            pltpu.SemaphoreType.DMA,
        ],
        compiler_params=pltpu.CompilerParams(
            dimension_semantics=("arbitrary",),
            vmem_limit_bytes=48 * 1024 * 1024,
        ),
        name="wave_decoder_step",
    )(x, num, cat32, embt, W_in, binr, q2, wc0, wc1, bc,
      W_res, brt, W_skip, bst, wo1, bo1, W_o2, bo2)

    return out.reshape(B, 1, 1), jnp.transpose(newq, (0, 3, 2, 1))


# speedup vs baseline: 1.0508x; 1.0508x over previous
import jax
import jax.numpy as jnp
from jax import lax
from jax.experimental import pallas as pl
from jax.experimental.pallas import tpu as pltpu

_NBLK = 24
_DILC = 8
_BT = 128


def _decoder_kernel(x_ref, num_ref, cat_ref, embt_ref, win_ref, bin_ref,
                    q_ref, wc0_ref, wc1_ref, bc_ref, wrt_ref, brt_ref,
                    wst_ref, bst_ref, wo1_ref, bo1_ref, wo2_ref, bo2_ref,
                    out_ref, newq_ref, cur_ref, acc_ref):
    i = pl.program_id(1)
    f32 = jnp.float32
    BT = x_ref.shape[2]
    C = cur_ref.shape[0]
    Tq = q_ref.shape[2] // C

    @pl.when(i == 0)
    def _init():
        xT = x_ref[0]
        numT = num_ref[0]
        catT = cat_ref[0]
        ohT = (lax.broadcasted_iota(jnp.int32, (1000, 1), 0) == catT).astype(f32)
        embT = jnp.dot(embt_ref[...], ohT, preferred_element_type=f32)
        w = win_ref[...]
        cur0 = (w[:, 0:1] * xT
                + jnp.dot(w[:, 1:9], numT, preferred_element_type=f32)
                + jnp.dot(w[:, 9:25], embT, preferred_element_type=f32)
                + bin_ref[...])
        cur_ref[...] = cur0
        acc_ref[...] = jnp.zeros_like(acc_ref)

    qT = jnp.transpose(q_ref[0], (1, 0))
    newq_ref[0, 0:Tq] = jnp.transpose(qT.reshape(C, Tq, BT), (1, 0, 2))
    cur = cur_ref[...]
    newq_ref[0, Tq] = cur

    def _tap(d):
        return lambda: newq_ref[0, Tq - d]

    tapT = lax.switch(jnp.bitwise_and(i, _DILC - 1),
                      [_tap(1 << k) for k in range(_DILC)])

    zT = (jnp.dot(wc0_ref[0], tapT, preferred_element_type=f32)
          + jnp.dot(wc1_ref[0], cur, preferred_element_type=f32)
          + bc_ref[0])
    fz = jnp.tanh(zT[:C, :])
    gz = zT[C:, :]
    gatedT = fz / (1.0 + jnp.exp(-gz))

    skipT = jnp.dot(wst_ref[0], gatedT, preferred_element_type=f32) + bst_ref[0]
    acc_ref[...] += jnp.dot(wo1_ref[0], jnp.maximum(skipT, 0.0),
                            preferred_element_type=f32)
    cur_ref[...] = (jnp.dot(wrt_ref[0], gatedT, preferred_element_type=f32)
                    + brt_ref[0] + cur)

    @pl.when(i == _NBLK - 1)
    def _finalize():
        hT = jnp.maximum(acc_ref[...] + bo1_ref[...], 0.0)
        outT = jnp.dot(wo2_ref[...], hT, preferred_element_type=f32) + bo2_ref[...]
        out_ref[...] = outT[None]


def kernel(queues, x, num, cat, emb_table, W_in, b_in, W_conv, b_conv,
           W_res, b_res, W_skip, b_skip, W_o1, b_o1, W_o2, b_o2):
    B, C, Tq = queues.shape[1], queues.shape[2], queues.shape[3]
    S = W_skip.shape[1]
    nbt = B // _BT

    q2 = queues.reshape(_NBLK, B, C * Tq)
    xT = x[:, :, 0].T[None]
    numT = num[:, :, 0].T[None]
    catT = cat[:, :, 0].astype(jnp.int32).T[None]
    wc0 = W_conv[:, :, :, 0]
    wc1 = W_conv[:, :, :, 1]
    bc = b_conv[:, :, None]
    brt = b_res[:, :, None]
    bst = b_skip[:, :, None]
    wo1 = W_o1.reshape(128, _NBLK, S).transpose(1, 0, 2)
    binr = b_in[:, None]
    bo1 = b_o1[:, None]
    bo2 = b_o2[:, None]
    embt = emb_table.T

    bspec = pl.BlockSpec

    out, newq = pl.pallas_call(
        _decoder_kernel,
        grid=(nbt, _NBLK),
        in_specs=[
            bspec((1, 1, _BT), lambda b, i: (0, 0, b)),
            bspec((1, 8, _BT), lambda b, i: (0, 0, b)),
            bspec((1, 1, _BT), lambda b, i: (0, 0, b)),
            bspec((16, 1000), lambda b, i: (0, 0)),
            bspec((C, 25), lambda b, i: (0, 0)),
            bspec((C, 1), lambda b, i: (0, 0)),
            bspec((1, _BT, C * Tq), lambda b, i: (i, b, 0)),
            bspec((1, 2 * C, C), lambda b, i: (i, 0, 0)),
            bspec((1, 2 * C, C), lambda b, i: (i, 0, 0)),
            bspec((1, 2 * C, 1), lambda b, i: (i, 0, 0)),
            bspec((1, C, C), lambda b, i: (i, 0, 0)),
            bspec((1, C, 1), lambda b, i: (i, 0, 0)),
            bspec((1, S, C), lambda b, i: (i, 0, 0)),
            bspec((1, S, 1), lambda b, i: (i, 0, 0)),
            bspec((1, 128, S), lambda b, i: (i, 0, 0)),
            bspec((128, 1), lambda b, i: (0, 0)),
            bspec((1, 128), lambda b, i: (0, 0)),
            bspec((1, 1), lambda b, i: (0, 0)),
        ],
        out_specs=[
            bspec((1, 1, _BT), lambda b, i: (0, 0, b)),
            bspec((1, Tq + 1, C, _BT), lambda b, i: (i, 0, 0, b)),
        ],
        out_shape=[
            jax.ShapeDtypeStruct((1, 1, B), jnp.float32),
            jax.ShapeDtypeStruct((_NBLK, Tq + 1, C, B), jnp.float32),
        ],
        scratch_shapes=[
            pltpu.VMEM((C, _BT), jnp.float32),
            pltpu.VMEM((128, _BT), jnp.float32),
        ],
        compiler_params=pltpu.CompilerParams(
            dimension_semantics=("parallel", "arbitrary"),
            vmem_limit_bytes=48 * 1024 * 1024,
        ),
        name="wave_decoder_step",
    )(xT, numT, catT, embt, W_in, binr, q2, wc0, wc1, bc,
      W_res, brt, W_skip, bst, wo1, bo1, W_o2, bo2)

    return out.reshape(B, 1, 1), jnp.transpose(newq, (0, 3, 2, 1))


# speedup vs baseline: 2.5913x; 2.4661x over previous
import jax
import jax.numpy as jnp
from jax import lax
from jax.experimental import pallas as pl
from jax.experimental.pallas import tpu as pltpu

_NBLK = 24
_DILC = 8


def _decoder_kernel(x_ref, num_ref, cat_ref, embt_ref, win_ref, bin_ref,
                    q_hbm, wc0_ref, wc1_ref, bc_ref, wrt_ref, brt_ref,
                    wst_ref, bst_ref, wo1_ref, bo1_ref, wo2_ref, bo2_ref,
                    out_ref, newq_ref, vin_ref, cur_ref, acc_ref, sems):
    i = pl.program_id(0)
    f32 = jnp.float32
    B = x_ref.shape[2]
    C = cur_ref.shape[0]
    Tq = q_hbm.shape[3]
    s = jnp.bitwise_and(i, 1)

    def _in_copy(blk, slot, c):
        return pltpu.make_async_copy(q_hbm.at[blk, :, c, :],
                                     vin_ref.at[slot, c], sems.at[slot, c])

    @pl.when(i == 0)
    def _first_fetch():
        for c in range(C):
            _in_copy(0, 0, c).start()

    @pl.when(i + 1 < _NBLK)
    def _prefetch():
        for c in range(C):
            _in_copy(i + 1, 1 - s, c).start()

    for c in range(C):
        _in_copy(i, s, c).wait()

    @pl.when(i == 0)
    def _init():
        xT = x_ref[0]
        numT = num_ref[0]
        catT = cat_ref[0]
        ohT = (lax.broadcasted_iota(jnp.int32, (1000, 1), 0) == catT).astype(f32)
        embT = jnp.dot(embt_ref[...], ohT, preferred_element_type=f32)
        w = win_ref[...]
        cur0 = (w[:, 0:1] * xT
                + jnp.dot(w[:, 1:9], numT, preferred_element_type=f32)
                + jnp.dot(w[:, 9:25], embT, preferred_element_type=f32)
                + bin_ref[...])
        cur_ref[...] = cur0
        acc_ref[...] = jnp.zeros_like(acc_ref)

    vt = jnp.transpose(vin_ref[s], (0, 2, 1))
    newq_ref[0, 0:Tq] = jnp.transpose(vt, (1, 0, 2))
    cur = cur_ref[...]
    newq_ref[0, Tq] = cur

    def _tap(d):
        return lambda: newq_ref[0, Tq - d]

    tapT = lax.switch(jnp.bitwise_and(i, _DILC - 1),
                      [_tap(1 << k) for k in range(_DILC)])

    zT = (jnp.dot(wc0_ref[0], tapT, preferred_element_type=f32)
          + jnp.dot(wc1_ref[0], cur, preferred_element_type=f32)
          + bc_ref[0])
    fz = jnp.tanh(zT[:C, :])
    gz = zT[C:, :]
    gatedT = fz / (1.0 + jnp.exp(-gz))

    skipT = jnp.dot(wst_ref[0], gatedT, preferred_element_type=f32) + bst_ref[0]
    acc_ref[...] += jnp.dot(wo1_ref[0], jnp.maximum(skipT, 0.0),
                            preferred_element_type=f32)
    cur_ref[...] = (jnp.dot(wrt_ref[0], gatedT, preferred_element_type=f32)
                    + brt_ref[0] + cur)

    @pl.when(i == _NBLK - 1)
    def _finalize():
        hT = jnp.maximum(acc_ref[...] + bo1_ref[...], 0.0)
        outT = jnp.dot(wo2_ref[...], hT, preferred_element_type=f32) + bo2_ref[...]
        out_ref[...] = outT[None]


def kernel(queues, x, num, cat, emb_table, W_in, b_in, W_conv, b_conv,
           W_res, b_res, W_skip, b_skip, W_o1, b_o1, W_o2, b_o2):
    B, C, Tq = queues.shape[1], queues.shape[2], queues.shape[3]
    S = W_skip.shape[1]

    xT = x[:, :, 0].T[None]
    numT = num[:, :, 0].T[None]
    catT = cat[:, :, 0].astype(jnp.int32).T[None]
    wc0 = W_conv[:, :, :, 0]
    wc1 = W_conv[:, :, :, 1]
    bc = b_conv[:, :, None]
    brt = b_res[:, :, None]
    bst = b_skip[:, :, None]
    wo1 = W_o1.reshape(128, _NBLK, S).transpose(1, 0, 2)
    binr = b_in[:, None]
    bo1 = b_o1[:, None]
    bo2 = b_o2[:, None]
    embt = emb_table.T

    bspec = pl.BlockSpec

    out, newq = pl.pallas_call(
        _decoder_kernel,
        grid=(_NBLK,),
        in_specs=[
            bspec((1, 1, B), lambda i: (0, 0, 0)),
            bspec((1, 8, B), lambda i: (0, 0, 0)),
            bspec((1, 1, B), lambda i: (0, 0, 0)),
            bspec((16, 1000), lambda i: (0, 0)),
            bspec((C, 25), lambda i: (0, 0)),
            bspec((C, 1), lambda i: (0, 0)),
            bspec(memory_space=pl.ANY),
            bspec((1, 2 * C, C), lambda i: (i, 0, 0)),
            bspec((1, 2 * C, C), lambda i: (i, 0, 0)),
            bspec((1, 2 * C, 1), lambda i: (i, 0, 0)),
            bspec((1, C, C), lambda i: (i, 0, 0)),
            bspec((1, C, 1), lambda i: (i, 0, 0)),
            bspec((1, S, C), lambda i: (i, 0, 0)),
            bspec((1, S, 1), lambda i: (i, 0, 0)),
            bspec((1, 128, S), lambda i: (i, 0, 0)),
            bspec((128, 1), lambda i: (0, 0)),
            bspec((1, 128), lambda i: (0, 0)),
            bspec((1, 1), lambda i: (0, 0)),
        ],
        out_specs=[
            bspec((1, 1, B), lambda i: (0, 0, 0)),
            bspec((1, Tq + 1, C, B), lambda i: (i, 0, 0, 0)),
        ],
        out_shape=[
            jax.ShapeDtypeStruct((1, 1, B), jnp.float32),
            jax.ShapeDtypeStruct((_NBLK, Tq + 1, C, B), jnp.float32),
        ],
        scratch_shapes=[
            pltpu.VMEM((2, C, B, Tq), jnp.float32),
            pltpu.VMEM((C, B), jnp.float32),
            pltpu.VMEM((128, B), jnp.float32),
            pltpu.SemaphoreType.DMA((2, C)),
        ],
        compiler_params=pltpu.CompilerParams(
            dimension_semantics=("arbitrary",),
            vmem_limit_bytes=48 * 1024 * 1024,
        ),
        name="wave_decoder_step",
    )(xT, numT, catT, embt, W_in, binr, queues, wc0, wc1, bc,
      W_res, brt, W_skip, bst, wo1, bo1, W_o2, bo2)

    return out.reshape(B, 1, 1), jnp.transpose(newq, (0, 3, 2, 1))


# speedup vs baseline: 2.6468x; 1.0214x over previous
import jax
import jax.numpy as jnp
from jax import lax
from jax.experimental import pallas as pl
from jax.experimental.pallas import tpu as pltpu

_NBLK = 24
_DILC = 8


def _decoder_kernel(x_ref, num_ref, cat_ref, embt_ref, win_ref, bin_ref,
                    q_hbm, wc0_ref, wc1_ref, bc_ref, wrt_ref, brt_ref,
                    wst_ref, bst_ref, wo1_ref, bo1_ref, wo2_ref, bo2_ref,
                    out_ref, newq_ref, vin_ref, cur_ref, acc_ref, sems):
    i = pl.program_id(0)
    f32 = jnp.float32
    B = x_ref.shape[2]
    C = cur_ref.shape[0]
    Tq = q_hbm.shape[3]
    s = jnp.remainder(i, 3)

    def _in_copy(blk, slot, c):
        return pltpu.make_async_copy(q_hbm.at[blk, :, c, :],
                                     vin_ref.at[slot, c], sems.at[slot, c])

    @pl.when(i == 0)
    def _first_fetch():
        for c in range(C):
            _in_copy(0, 0, c).start()
            _in_copy(1, 1, c).start()

    @pl.when(i + 2 < _NBLK)
    def _prefetch():
        for c in range(C):
            _in_copy(i + 2, jnp.remainder(i + 2, 3), c).start()

    for c in range(C):
        _in_copy(i, s, c).wait()

    @pl.when(i == 0)
    def _init():
        xT = x_ref[0]
        numT = num_ref[0]
        catT = cat_ref[0]
        ohT = (lax.broadcasted_iota(jnp.int32, (1000, 1), 0) == catT).astype(f32)
        embT = jnp.dot(embt_ref[...], ohT, preferred_element_type=f32)
        w = win_ref[...]
        cur0 = (w[:, 0:1] * xT
                + jnp.dot(w[:, 1:9], numT, preferred_element_type=f32)
                + jnp.dot(w[:, 9:25], embT, preferred_element_type=f32)
                + bin_ref[...])
        cur_ref[...] = cur0
        acc_ref[...] = jnp.zeros_like(acc_ref)

    vt = jnp.transpose(vin_ref[s], (0, 2, 1))
    newq_ref[0, 0:Tq] = jnp.transpose(vt, (1, 0, 2))
    cur = cur_ref[...]
    newq_ref[0, Tq] = cur

    def _tap(d):
        return lambda: newq_ref[0, Tq - d]

    tapT = lax.switch(jnp.bitwise_and(i, _DILC - 1),
                      [_tap(1 << k) for k in range(_DILC)])

    zT = (jnp.dot(wc0_ref[0], tapT, preferred_element_type=f32)
          + jnp.dot(wc1_ref[0], cur, preferred_element_type=f32)
          + bc_ref[0])
    fz = jnp.tanh(zT[:C, :])
    gz = zT[C:, :]
    gatedT = fz / (1.0 + jnp.exp(-gz))

    skipT = jnp.dot(wst_ref[0], gatedT, preferred_element_type=f32) + bst_ref[0]
    acc_ref[...] += jnp.dot(wo1_ref[0], jnp.maximum(skipT, 0.0),
                            preferred_element_type=f32)
    cur_ref[...] = (jnp.dot(wrt_ref[0], gatedT, preferred_element_type=f32)
                    + brt_ref[0] + cur)

    @pl.when(i == _NBLK - 1)
    def _finalize():
        hT = jnp.maximum(acc_ref[...] + bo1_ref[...], 0.0)
        outT = jnp.dot(wo2_ref[...], hT, preferred_element_type=f32) + bo2_ref[...]
        out_ref[...] = outT[None]


def kernel(queues, x, num, cat, emb_table, W_in, b_in, W_conv, b_conv,
           W_res, b_res, W_skip, b_skip, W_o1, b_o1, W_o2, b_o2):
    B, C, Tq = queues.shape[1], queues.shape[2], queues.shape[3]
    S = W_skip.shape[1]

    xT = x[:, :, 0].T[None]
    numT = num[:, :, 0].T[None]
    catT = cat[:, :, 0].astype(jnp.int32).T[None]
    wc0 = W_conv[:, :, :, 0]
    wc1 = W_conv[:, :, :, 1]
    bc = b_conv[:, :, None]
    brt = b_res[:, :, None]
    bst = b_skip[:, :, None]
    wo1 = W_o1.reshape(128, _NBLK, S).transpose(1, 0, 2)
    binr = b_in[:, None]
    bo1 = b_o1[:, None]
    bo2 = b_o2[:, None]
    embt = emb_table.T

    bspec = pl.BlockSpec

    out, newq = pl.pallas_call(
        _decoder_kernel,
        grid=(_NBLK,),
        in_specs=[
            bspec((1, 1, B), lambda i: (0, 0, 0)),
            bspec((1, 8, B), lambda i: (0, 0, 0)),
            bspec((1, 1, B), lambda i: (0, 0, 0)),
            bspec((16, 1000), lambda i: (0, 0)),
            bspec((C, 25), lambda i: (0, 0)),
            bspec((C, 1), lambda i: (0, 0)),
            bspec(memory_space=pl.ANY),
            bspec((1, 2 * C, C), lambda i: (i, 0, 0)),
            bspec((1, 2 * C, C), lambda i: (i, 0, 0)),
            bspec((1, 2 * C, 1), lambda i: (i, 0, 0)),
            bspec((1, C, C), lambda i: (i, 0, 0)),
            bspec((1, C, 1), lambda i: (i, 0, 0)),
            bspec((1, S, C), lambda i: (i, 0, 0)),
            bspec((1, S, 1), lambda i: (i, 0, 0)),
            bspec((1, 128, S), lambda i: (i, 0, 0)),
            bspec((128, 1), lambda i: (0, 0)),
            bspec((1, 128), lambda i: (0, 0)),
            bspec((1, 1), lambda i: (0, 0)),
        ],
        out_specs=[
            bspec((1, 1, B), lambda i: (0, 0, 0)),
            bspec((1, Tq + 1, C, B), lambda i: (i, 0, 0, 0)),
        ],
        out_shape=[
            jax.ShapeDtypeStruct((1, 1, B), jnp.float32),
            jax.ShapeDtypeStruct((_NBLK, Tq + 1, C, B), jnp.float32),
        ],
        scratch_shapes=[
            pltpu.VMEM((3, C, B, Tq), jnp.float32),
            pltpu.VMEM((C, B), jnp.float32),
            pltpu.VMEM((128, B), jnp.float32),
            pltpu.SemaphoreType.DMA((3, C)),
        ],
        compiler_params=pltpu.CompilerParams(
            dimension_semantics=("arbitrary",),
            vmem_limit_bytes=48 * 1024 * 1024,
        ),
        name="wave_decoder_step",
    )(xT, numT, catT, embt, W_in, binr, queues, wc0, wc1, bc,
      W_res, brt, W_skip, bst, wo1, bo1, W_o2, bo2)

    return out.reshape(B, 1, 1), jnp.transpose(newq, (0, 3, 2, 1))
